# baseline (speedup 1.0000x reference)
import functools

import jax
import jax.numpy as jnp
from jax import lax
from jax.experimental import pallas as pl
from jax.experimental.pallas import tpu as pltpu
from jax.experimental.pallas import tpu_sc as plsc

NB = 5
NN = 10000
NT = 50000
NE = 800000
NL = 128
NH = 64
NPAD = 51200
ROWS = 6272
HALF = ROWS // 2
TROWS = ROWS // 16
NBLK = TROWS // 2
NC, NS = 2, 16
RB = 1600
GD = NPAD // RB


def _mesh():
    return plsc.VectorSubcoreMesh(
        core_axis_name="c", subcore_axis_name="s",
        num_cores=NC, num_subcores=NS)


DROWS = ROWS // NS
CROWS = HALF // NS


def _sc_prep_body(src_hbm, dst_hbm, batch_hbm, zn_hbm, zs_hbm,
                  s_out, pk_out,
                  sb0, sb1, sb2, sb3, db0, db1, db2, db3,
                  pv0, pv1, pv2, pv3, wv0, wv1, wv2, wv3,
                  fx0, fx1, fx2, fx3, ones, dgb, pb,
                  sem_i0, sem_i1, sem_i2, sem_i3, sem_g, sem_s,
                  acc_n, acc_s):
    c = lax.axis_index("c")
    s = lax.axis_index("s")
    pipes = ((sb0, db0, pv0, wv0, fx0, sem_i0),
             (sb1, db1, pv1, wv1, fx1, sem_i1),
             (sb2, db2, pv2, wv2, fx2, sem_i2),
             (sb3, db3, pv3, wv3, fx3, sem_i3))
    for i in range(8):
        ones[pl.ds(16 * i, 16)] = jnp.full((16,), 1, jnp.int32)

    based = s * DROWS

    def fire_d(b, dbuf, sem):
        pltpu.async_copy(dst_hbm.at[based + b], dbuf, sem)

    for k, (_, dbuf, _, _, _, sem) in enumerate(pipes):
        fire_d(k, dbuf, sem)
    pltpu.sync_copy(zn_hbm.at[pl.ds(s * 3200, 3200)],
                    acc_n.at[pl.ds(s * 3200, 3200)])
    pltpu.sync_copy(zs_hbm.at[pl.ds(s * 25600, 25600)],
                    acc_s.at[pl.ds(s * 25600, 25600)])
    plsc.subcore_barrier()

    def outer_d(i4, carry):
        scs = []
        for _, dbuf, _, _, _, sem in pipes:
            pltpu.make_async_copy(dst_hbm.at[0], dbuf, sem).wait()
            scs.append(pltpu.async_copy(ones, acc_n.at[dbuf], sem_s,
                                        add=True))
        for k, (_, dbuf, _, _, _, sem) in enumerate(pipes):
            scs[k].wait()
            fire_d(lax.rem(4 * i4 + k + 4, DROWS), dbuf, sem)
        return carry

    lax.fori_loop(0, DROWS // 4, outer_d, 0)
    for _, dbuf, _, _, _, sem in pipes:
        pltpu.make_async_copy(dst_hbm.at[0], dbuf, sem).wait()
    plsc.subcore_barrier()

    pltpu.sync_copy(acc_n.at[pl.ds(s * 3200, 3200)], dgb)
    pltpu.sync_copy(batch_hbm.at[pl.ds(s * 3200, 3200)], pb)

    def pack_step(i, carry):
        o = i * 16
        x = (dgb[pl.ds(o, 16)] + 1).astype(jnp.float32)
        ii = lax.bitcast_convert_type(x, jnp.int32)
        y = lax.bitcast_convert_type(
            jnp.int32(0x5F3759DF) - lax.shift_right_arithmetic(ii, 1),
            jnp.float32)
        for _ in range(3):
            y = y * (1.5 - 0.5 * x * y * y)
        dgb[pl.ds(o, 16)] = (
            (lax.bitcast_convert_type(y, jnp.int32) & jnp.int32(-8))
            | pb[pl.ds(o, 16)])
        return carry

    lax.fori_loop(0, 200, pack_step, 0)
    pltpu.sync_copy(dgb, acc_n.at[pl.ds(s * 3200, 3200)])

    @pl.when(s // 8 == c)
    def _():
        pltpu.sync_copy(dgb, pk_out.at[pl.ds(s * 3200, 3200)])
    plsc.subcore_barrier()

    basec = c * HALF + s * CROWS

    def fire_c(b, sbuf, dbuf, sem):
        r = basec + b
        pltpu.async_copy(src_hbm.at[r], sbuf, sem)
        pltpu.async_copy(dst_hbm.at[r], dbuf, sem)

    for k, (sbuf, dbuf, _, _, _, sem) in enumerate(pipes):
        fire_c(k, sbuf, dbuf, sem)

    def outer_c(i4, carry):
        for sbuf, dbuf, pv, _, _, sem in pipes:
            pltpu.make_async_copy(src_hbm.at[0], sbuf, sem).wait()
            pltpu.make_async_copy(dst_hbm.at[0], dbuf, sem).wait()
            pltpu.async_copy(acc_n.at[sbuf], pv, sem_g)
        scs = []
        for sbuf, dbuf, pv, wv, fx, sem in pipes:
            pltpu.make_async_copy(acc_n.at[sbuf], pv, sem_g).wait()
            for k in range(8):
                p16 = pv[pl.ds(16 * k, 16)]
                d16 = dbuf[pl.ds(16 * k, 16)]
                fx[pl.ds(16 * k, 16)] = d16 * 8 + (p16 & jnp.int32(7))
                wv[pl.ds(16 * k, 16)] = lax.bitcast_convert_type(
                    p16 & jnp.int32(-8), jnp.float32)
            scs.append(pltpu.async_copy(wv, acc_s.at[fx], sem_s,
                                        add=True))
        for k, (sbuf, dbuf, _, _, _, sem) in enumerate(pipes):
            scs[k].wait()
            fire_c(lax.rem(4 * i4 + k + 4, CROWS), sbuf, dbuf, sem)
        return carry

    lax.fori_loop(0, CROWS // 4, outer_c, 0)
    for sbuf, dbuf, _, _, _, sem in pipes:
        pltpu.make_async_copy(src_hbm.at[0], sbuf, sem).wait()
        pltpu.make_async_copy(dst_hbm.at[0], dbuf, sem).wait()
    plsc.subcore_barrier()
    pltpu.sync_copy(acc_s.at[pl.ds(s * 25600, 25600)],
                    s_out.at[c, pl.ds(s * 25600, 25600)])


def _sc_prep(src2, dst2, batch_pad, zeros_ni, zeros_s):
    f = pl.kernel(
        _sc_prep_body,
        out_type=(jax.ShapeDtypeStruct((NC, NPAD * 8), jnp.float32),
                  jax.ShapeDtypeStruct((NPAD,), jnp.int32)),
        mesh=_mesh(),
        scratch_types=(
            [pltpu.VMEM((128,), jnp.int32) for _ in range(8)]
            + [pltpu.VMEM((128,), jnp.int32) for _ in range(4)]
            + [pltpu.VMEM((128,), jnp.float32) for _ in range(4)]
            + [pltpu.VMEM((128,), jnp.int32) for _ in range(4)]
            + [pltpu.VMEM((128,), jnp.int32),
               pltpu.VMEM((3200,), jnp.int32),
               pltpu.VMEM((3200,), jnp.int32),
               pltpu.SemaphoreType.DMA,
               pltpu.SemaphoreType.DMA,
               pltpu.SemaphoreType.DMA,
               pltpu.SemaphoreType.DMA,
               pltpu.SemaphoreType.DMA,
               pltpu.SemaphoreType.DMA,
               pltpu.VMEM_SHARED((NPAD,), jnp.int32),
               pltpu.VMEM_SHARED((NPAD * 8,), jnp.float32)]))
    return f(src2, dst2, batch_pad, zeros_ni, zeros_s)


def _sc_conv2_body(src_hbm, dst_hbm, y_hbm, zeros_hbm, out_hbm,
                   si0, di0, si1, di1, si2, di2, si3, di3,
                   m0, m1, m2, m3,
                   sem_i0, sem_i1, sem_i2, sem_i3, sem_g, sem_s, acc):
    c = lax.axis_index("c")
    s = lax.axis_index("s")
    base = s * TROWS
    yc = y_hbm.at[c]
    pipes = ((si0, di0, m0, sem_i0), (si1, di1, m1, sem_i1),
             (si2, di2, m2, sem_i2), (si3, di3, m3, sem_i3))

    def fire_idx(b, sbuf, dbuf, sem):
        r = base + b
        pltpu.async_copy(src_hbm.at[r], sbuf, sem)
        pltpu.async_copy(dst_hbm.at[r], dbuf, sem)

    for k, (sbuf, dbuf, _, sem) in enumerate(pipes):
        fire_idx(k, sbuf, dbuf, sem)
    pltpu.sync_copy(zeros_hbm.at[pl.ds(s * 3200, 3200), :],
                    acc.at[pl.ds(s * 3200, 3200), :])
    plsc.subcore_barrier()

    def outer(i4, carry):
        for sbuf, dbuf, msg, sem in pipes:
            pltpu.make_async_copy(src_hbm.at[0], sbuf, sem).wait()
            pltpu.make_async_copy(dst_hbm.at[0], dbuf, sem).wait()
            pltpu.async_copy(yc.at[sbuf], msg, sem_g)
        scs = []
        for sbuf, dbuf, msg, sem in pipes:
            pltpu.make_async_copy(yc.at[sbuf], msg, sem_g).wait()
            scs.append(pltpu.async_copy(msg, acc.at[dbuf], sem_s,
                                        add=True))
        for k, (sbuf, dbuf, msg, sem) in enumerate(pipes):
            scs[k].wait()
            fire_idx(lax.rem(4 * i4 + k + 4, TROWS), sbuf, dbuf, sem)
        return carry

    lax.fori_loop(0, TROWS // 4, outer, 0)
    for sbuf, dbuf, _, sem in pipes:
        pltpu.make_async_copy(src_hbm.at[0], sbuf, sem).wait()
        pltpu.make_async_copy(dst_hbm.at[0], dbuf, sem).wait()
    plsc.subcore_barrier()
    pltpu.sync_copy(acc.at[pl.ds(s * 3200, 3200), :],
                    out_hbm.at[c, pl.ds(s * 3200, 3200), :])


def _sc_conv2(src2, dst2, y2, zeros_m):
    f = pl.kernel(
        _sc_conv2_body,
        out_type=jax.ShapeDtypeStruct((NC, NPAD, 32), jnp.float32),
        mesh=_mesh(),
        compiler_params=pltpu.CompilerParams(use_tc_tiling_on_sc=False),
        scratch_types=[
            pltpu.VMEM((128,), jnp.int32),
            pltpu.VMEM((128,), jnp.int32),
            pltpu.VMEM((128,), jnp.int32),
            pltpu.VMEM((128,), jnp.int32),
            pltpu.VMEM((128,), jnp.int32),
            pltpu.VMEM((128,), jnp.int32),
            pltpu.VMEM((128,), jnp.int32),
            pltpu.VMEM((128,), jnp.int32),
            pltpu.VMEM((128, 32), jnp.float32),
            pltpu.VMEM((128, 32), jnp.float32),
            pltpu.VMEM((128, 32), jnp.float32),
            pltpu.VMEM((128, 32), jnp.float32),
            pltpu.SemaphoreType.DMA,
            pltpu.SemaphoreType.DMA,
            pltpu.SemaphoreType.DMA,
            pltpu.SemaphoreType.DMA,
            pltpu.SemaphoreType.DMA,
            pltpu.SemaphoreType.DMA,
            pltpu.VMEM_SHARED((NPAD, 32), jnp.float32),
        ])
    return f(src2, dst2, y2, zeros_m)


def _tc_dense_body(S_ref, pk_ref, z_ref, Wz_ref, bz_ref,
                   W1_ref, b1_ref, W2_ref, b2_ref,
                   y2_ref, q_ref):
    bf = jnp.bfloat16
    f32 = jnp.float32
    zn = lax.dot_general(z_ref[...].astype(bf), Wz_ref[...].astype(bf),
                         (((1,), (1,)), ((), ())),
                         preferred_element_type=f32) + bz_ref[...]
    xw1d = lax.dot_general(zn.astype(bf), W1_ref[...].astype(bf),
                           (((1,), (1,)), ((), ())),
                           preferred_element_type=f32)
    S = S_ref[0] + S_ref[1]
    pk = pk_ref[0, 0, :]
    dv = lax.bitcast_convert_type(pk & jnp.int32(-8), f32)
    bt = pk & jnp.int32(7)
    oh = (bt[:, None] == lax.broadcasted_iota(jnp.int32, (RB, 5), 1)
          ).astype(jnp.float32)
    S5 = S[:, :5] + dv[:, None] * oh
    t = S5[:, 0:1] * xw1d[0:1, :]
    for b in range(1, 5):
        t = t + S5[:, b:b + 1] * xw1d[b:b + 1, :]
    out1 = dv[:, None] * t + b1_ref[...]
    h1 = jnp.maximum(out1, 0.0)
    xw2 = lax.dot_general(h1.astype(bf), W2_ref[...].astype(bf),
                          (((1,), (1,)), ((), ())),
                          preferred_element_type=f32)
    y = dv[:, None] * xw2
    q_ref[...] = dv[:, None] * y + b2_ref[...]
    y2_ref[0] = y[:, :32]
    y2_ref[1] = y[:, 32:]


def _tc_dense(S3, pk3, z, Wz, bz, W1, b1, W2, b2):
    return pl.pallas_call(
        _tc_dense_body,
        grid=(GD,),
        in_specs=[
            pl.BlockSpec((2, RB, 8), lambda i: (0, i, 0)),
            pl.BlockSpec((1, 1, RB), lambda i: (i, 0, 0)),
            pl.BlockSpec((NB, NL), lambda i: (0, 0)),
            pl.BlockSpec((NH, NL), lambda i: (0, 0)),
            pl.BlockSpec((NH,), lambda i: (0,)),
            pl.BlockSpec((NH, NH), lambda i: (0, 0)),
            pl.BlockSpec((NH,), lambda i: (0,)),
            pl.BlockSpec((NH, NH), lambda i: (0, 0)),
            pl.BlockSpec((NH,), lambda i: (0,)),
        ],
        out_specs=[
            pl.BlockSpec((2, RB, 32), lambda i: (0, i, 0)),
            pl.BlockSpec((RB, NH), lambda i: (i, 0)),
        ],
        out_shape=[
            jax.ShapeDtypeStruct((2, NPAD, 32), jnp.float32),
            jax.ShapeDtypeStruct((NPAD, NH), jnp.float32),
        ],
    )(S3, pk3, z, Wz, bz, W1, b1, W2, b2)


def _tc_final_body(seg_ref, q_ref, pk_ref, Wo_ref, bo_ref, o_ref):
    pk = pk_ref[0, 0, :]
    dv = lax.bitcast_convert_type(pk & jnp.int32(-8), jnp.float32)
    wo = Wo_ref[...]
    q = q_ref[...]
    h0 = jnp.maximum(dv[:, None] * seg_ref[0] + q[:, :32], 0.0)
    h1 = jnp.maximum(dv[:, None] * seg_ref[1] + q[:, 32:], 0.0)
    bf = jnp.bfloat16
    f32 = jnp.float32
    lin = (jnp.dot(h0.astype(bf), wo[0, :32].astype(bf),
                   preferred_element_type=f32)
           + jnp.dot(h1.astype(bf), wo[0, 32:].astype(bf),
                     preferred_element_type=f32) + bo_ref[0])
    o_ref[...] = jnp.tanh(lin).reshape(1, 1, RB)


def _tc_final(seg2, q, pk3, Wo, bo):
    return pl.pallas_call(
        _tc_final_body,
        grid=(GD,),
        in_specs=[
            pl.BlockSpec((2, RB, 32), lambda i: (0, i, 0)),
            pl.BlockSpec((RB, NH), lambda i: (i, 0)),
            pl.BlockSpec((1, 1, RB), lambda i: (i, 0, 0)),
            pl.BlockSpec((1, NH), lambda i: (0, 0)),
            pl.BlockSpec((1,), lambda i: (0,)),
        ],
        out_specs=pl.BlockSpec((1, 1, RB), lambda i: (i, 0, 0)),
        out_shape=jax.ShapeDtypeStruct((GD, 1, RB), jnp.float32),
    )(seg2, q, pk3, Wo, bo)


def kernel(z, edge_index, batch, Wz, bz, W1, b1, W2, b2, Wo, bo):
    npad_e = ROWS * 128 - NE
    pad = jnp.full((npad_e,), NT, jnp.int32)
    src2 = jnp.concatenate([edge_index[0], pad]).reshape(ROWS, 128)
    dst2 = jnp.concatenate([edge_index[1], pad]).reshape(ROWS, 128)
    zeros_ni = jnp.zeros((NPAD,), jnp.int32)
    zeros_s = jnp.zeros((NPAD * 8,), jnp.float32)
    zeros_m = jnp.zeros((NPAD, 32), jnp.float32)
    batch_pad = jnp.concatenate(
        [batch, jnp.zeros((NPAD - NT,), jnp.int32)])

    S2, packed = _sc_prep(src2, dst2, batch_pad, zeros_ni, zeros_s)
    S3 = S2.reshape(NC, NPAD, 8)
    pk3 = packed.reshape(GD, 1, RB)
    y2, q = _tc_dense(S3, pk3, z, Wz, bz, W1, b1, W2, b2)
    seg2 = _sc_conv2(src2, dst2, y2, zeros_m)
    spin3 = _tc_final(seg2, q, pk3, Wo, bo)
    return spin3.reshape(NPAD)[:NT].reshape(NB, NN)
